# baseline (speedup 1.0000x reference)
import functools

import jax
import jax.numpy as jnp
from jax import lax
from jax.experimental import pallas as pl
from jax.experimental.pallas import tpu as pltpu
from jax.experimental.pallas import tpu_sc as plsc

B = 16384
D_ITEM = 32
D_GROUP = 16
H1, H2 = 128, 64

NC, NS = 2, 16
NW = NC * NS
BPW = B // NW

BLK = 2048


def _gather_body(pc_hbm, gh_hbm, gn_hbm, item_t, gh_t, gn_t,
                 item_out, gh_out, gn_out,
                 pc_v, gh_v, gn_v, item_rows, gh_rows, gn_rows,
                 sem1, sem2, sem3):
    wid = lax.axis_index("s") * NC + lax.axis_index("c")
    base = wid * BPW
    pltpu.sync_copy(pc_hbm.at[pl.ds(base, BPW)], pc_v)
    pltpu.sync_copy(gh_hbm.at[pl.ds(base, BPW)], gh_v)
    pltpu.sync_copy(gn_hbm.at[pl.ds(base, BPW)], gn_v)
    c1 = pltpu.async_copy(item_t.at[pc_v], item_rows, sem1)
    c2 = pltpu.async_copy(gh_t.at[gh_v], gh_rows, sem2)
    c3 = pltpu.async_copy(gn_t.at[gn_v], gn_rows, sem3)
    c1.wait()
    c2.wait()
    c3.wait()
    pltpu.sync_copy(item_rows, item_out.at[pl.ds(base, BPW)])
    pltpu.sync_copy(gh_rows, gh_out.at[pl.ds(base, BPW)])
    pltpu.sync_copy(gn_rows, gn_out.at[pl.ds(base, BPW)])


def _sc_gather():
    return pl.kernel(
        _gather_body,
        out_type=[
            jax.ShapeDtypeStruct((B, D_ITEM), jnp.float32),
            jax.ShapeDtypeStruct((B, D_GROUP), jnp.float32),
            jax.ShapeDtypeStruct((B, D_GROUP), jnp.float32),
        ],
        mesh=plsc.VectorSubcoreMesh(
            core_axis_name="c", subcore_axis_name="s",
            num_cores=NC, num_subcores=NS),
        compiler_params=pltpu.CompilerParams(use_tc_tiling_on_sc=False),
        scratch_types=[
        pltpu.VMEM((BPW,), jnp.int32),
        pltpu.VMEM((BPW,), jnp.int32),
        pltpu.VMEM((BPW,), jnp.int32),
        pltpu.VMEM((BPW, D_ITEM), jnp.float32),
        pltpu.VMEM((BPW, D_GROUP), jnp.float32),
        pltpu.VMEM((BPW, D_GROUP), jnp.float32),
            pltpu.SemaphoreType.DMA,
            pltpu.SemaphoreType.DMA,
            pltpu.SemaphoreType.DMA,
        ],
    )


def _mlp_body(item_ref, gh_ref, gn_ref, pb_ref,
              w1a_ref, w1b_ref, w1c_ref, w1pb_ref, b1_ref,
              w2_ref, b2_ref, wp_ref, bp_ref, out_ref):
    h = jnp.dot(item_ref[...], w1a_ref[...], preferred_element_type=jnp.float32)
    h += jnp.dot(gh_ref[...], w1b_ref[...], preferred_element_type=jnp.float32)
    h += jnp.dot(gn_ref[...], w1c_ref[...], preferred_element_type=jnp.float32)
    h += jnp.dot(pb_ref[...], w1pb_ref[...], preferred_element_type=jnp.float32)
    h = jnp.maximum(h + b1_ref[...], 0.0)
    h = jnp.maximum(
        jnp.dot(h, w2_ref[...], preferred_element_type=jnp.float32) + b2_ref[...],
        0.0)
    out_ref[...] = (
        jnp.dot(h, wp_ref[...], preferred_element_type=jnp.float32) + bp_ref[...])


def _mlp(item_vec, gh_vec, gn_vec, pb, W1a, W1b, W1c, W1pb, b1, W2, b2, Wp, bp,
         interpret=False):
    row = lambda i: (i, 0)
    rep = lambda i: (0, 0)
    return pl.pallas_call(
        _mlp_body,
        grid=(B // BLK,),
        in_specs=[
            pl.BlockSpec((BLK, D_ITEM), row),
            pl.BlockSpec((BLK, D_GROUP), row),
            pl.BlockSpec((BLK, D_GROUP), row),
            pl.BlockSpec((BLK, 2), row),
            pl.BlockSpec((D_ITEM, H1), rep),
            pl.BlockSpec((D_GROUP, H1), rep),
            pl.BlockSpec((D_GROUP, H1), rep),
            pl.BlockSpec((2, H1), rep),
            pl.BlockSpec((1, H1), rep),
            pl.BlockSpec((H1, H2), rep),
            pl.BlockSpec((1, H2), rep),
            pl.BlockSpec((H2, H2), rep),
            pl.BlockSpec((1, H2), rep),
        ],
        out_specs=pl.BlockSpec((BLK, H2), row),
        out_shape=jax.ShapeDtypeStruct((B, H2), jnp.float32),
        interpret=interpret,
    )(item_vec, gh_vec, gn_vec, pb, W1a, W1b, W1c, W1pb, b1, W2, b2, Wp, bp)


def kernel(ProductCode, ProductGroupHeader, ProductGroupName, Price, IsBestSeller,
           item_table, gh_table, gn_table, W1, b1, W2, b2, Wp, bp):
    item_vec, gh_vec, gn_vec = _sc_gather()(
        ProductCode, ProductGroupHeader, ProductGroupName,
        item_table, gh_table, gn_table)
    pb = jnp.concatenate(
        [Price.astype(jnp.float32)[:, None],
         IsBestSeller.astype(jnp.float32)[:, None]], axis=1)
    return _mlp(
        item_vec, gh_vec, gn_vec, pb,
        W1[:D_ITEM], W1[D_ITEM:D_ITEM + D_GROUP],
        W1[D_ITEM + D_GROUP:D_ITEM + 2 * D_GROUP], W1[D_ITEM + 2 * D_GROUP:],
        b1[None, :], W2, b2[None, :], Wp, bp[None, :])
